# baseline (speedup 1.0000x reference)
import functools

import jax
import jax.numpy as jnp
from jax import lax
from jax.experimental import pallas as pl
from jax.experimental.pallas import tpu as pltpu
from jax.experimental.pallas import tpu_sc as plsc

N = 10000
E = 320000
D = 128
H = 128
C = 10
B = 64
LAYERS = 3

NC = 2
NS = 16
NW = NC * NS
EPW = E // NW
K = 64
EPWP = 10240
NCHUNK = EPWP // K
RPS = 632
NP = NS * RPS


def _segsum_body(h_hbm, src_hbm, dst_hbm, zeros_hbm, out_hbm,
                 sidx_v, didx_v, rows_v, agg_s, sem):
    c = lax.axis_index("c")
    s = lax.axis_index("s")
    w = c * NS + s

    pltpu.sync_copy(zeros_hbm, agg_s.at[pl.ds(s * RPS, RPS)])
    pltpu.sync_copy(src_hbm.at[w], sidx_v)
    pltpu.sync_copy(dst_hbm.at[w], didx_v)

    plsc.subcore_barrier()

    def eloop(j, carry):
        pltpu.async_copy(h_hbm.at[sidx_v.at[j]], rows_v, sem).wait()
        pltpu.sync_copy(rows_v, agg_s.at[didx_v.at[j]], add=True)
        return carry

    lax.fori_loop(0, NCHUNK, eloop, 0)

    plsc.subcore_barrier()

    pltpu.sync_copy(agg_s.at[pl.ds(s * RPS, RPS)],
                    out_hbm.at[c].at[pl.ds(s * RPS, RPS)])


_segsum = pl.kernel(
    _segsum_body,
    out_type=jax.ShapeDtypeStruct((NC, NP, D), jnp.float32),
    mesh=plsc.VectorSubcoreMesh(core_axis_name="c", subcore_axis_name="s"),
    scratch_types=[
        pltpu.VMEM((NCHUNK, K), jnp.int32),
        pltpu.VMEM((NCHUNK, K), jnp.int32),
        pltpu.VMEM((K, D), jnp.float32),
        pltpu.VMEM_SHARED((NP, D), jnp.float32),
        pltpu.SemaphoreType.DMA,
    ],
)


RBLK = 400
NBLK = N // RBLK

_DN_T = (((1,), (1,)), ((), ()))
_HI = lax.Precision.HIGHEST


def _conv_body(residual, aggs_ref, h_ref, wrel_ref, brel_ref, wroot_ref,
               out_ref):
    a = aggs_ref[0] + aggs_ref[1]
    t = lax.dot_general(a, wrel_ref[...], _DN_T,
                        preferred_element_type=jnp.float32, precision=_HI)
    r = lax.dot_general(h_ref[...], wroot_ref[...], _DN_T,
                        preferred_element_type=jnp.float32, precision=_HI)
    o = jnp.maximum(t + brel_ref[...] + r, 0.0)
    if residual:
        o = o + h_ref[...]
    out_ref[...] = o


def _make_conv(residual):
    return pl.pallas_call(
        functools.partial(_conv_body, residual),
        grid=(NBLK,),
        in_specs=[
            pl.BlockSpec((NC, RBLK, D), lambda i: (0, i, 0)),
            pl.BlockSpec((RBLK, D), lambda i: (i, 0)),
            pl.BlockSpec((H, D), lambda i: (0, 0)),
            pl.BlockSpec((1, H), lambda i: (0, 0)),
            pl.BlockSpec((H, D), lambda i: (0, 0)),
        ],
        out_specs=pl.BlockSpec((RBLK, H), lambda i: (i, 0)),
        out_shape=jax.ShapeDtypeStruct((N, H), jnp.float32),
    )


_conv0 = _make_conv(False)
_convr = _make_conv(True)


def _pool_body(h_ref, b3_ref, wlin_ref, blin_ref, out_ref, acc, cnt):
    i = pl.program_id(0)

    @pl.when(i == 0)
    def _():
        acc[...] = jnp.zeros_like(acc)
        cnt[...] = jnp.zeros_like(cnt)

    bid = b3_ref[0]
    rows = lax.broadcasted_iota(jnp.int32, (B, RBLK), 0)
    m = (bid == rows).astype(jnp.float32)
    acc[...] += lax.dot_general(m, h_ref[...], (((1,), (0,)), ((), ())),
                                preferred_element_type=jnp.float32,
                                precision=_HI)
    cnt[...] += jnp.broadcast_to(jnp.sum(m, axis=1, keepdims=True), (B, H))

    @pl.when(i == NBLK - 1)
    def _():
        pooled = acc[...] / jnp.maximum(cnt[...], 1.0)
        out_ref[...] = lax.dot_general(
            pooled, wlin_ref[...], _DN_T,
            preferred_element_type=jnp.float32, precision=_HI) + blin_ref[...]


_pool = pl.pallas_call(
    _pool_body,
    grid=(NBLK,),
    in_specs=[
        pl.BlockSpec((RBLK, H), lambda i: (i, 0)),
        pl.BlockSpec((1, 1, RBLK), lambda i: (i, 0, 0)),
        pl.BlockSpec((C, H), lambda i: (0, 0)),
        pl.BlockSpec((1, C), lambda i: (0, 0)),
    ],
    out_specs=pl.BlockSpec((B, C), lambda i: (0, 0)),
    out_shape=jax.ShapeDtypeStruct((B, C), jnp.float32),
    scratch_shapes=[
        pltpu.VMEM((B, H), jnp.float32),
        pltpu.VMEM((B, H), jnp.float32),
    ],
)


def kernel(x, edge_index, batch, W_rel1, b_rel1, W_root1, W_rel2, b_rel2,
           W_root2, W_lin, b_lin):
    src = edge_index[0].reshape(NW, EPW)
    dst = edge_index[1].reshape(NW, EPW)
    pad = EPWP - EPW
    srcp = jnp.concatenate(
        [src, jnp.zeros((NW, pad), jnp.int32)],
        axis=1).reshape(NW, NCHUNK, K)
    dstp = jnp.concatenate(
        [dst, jnp.full((NW, pad), N, jnp.int32)],
        axis=1).reshape(NW, NCHUNK, K)
    zrows = jnp.zeros((RPS, D), jnp.float32)
    batch3 = batch.reshape(NBLK, 1, RBLK)
    b_rel1_2 = b_rel1.reshape(1, H)
    b_rel2_2 = b_rel2.reshape(1, H)
    b_lin_2 = b_lin.reshape(1, C)

    aggs = _segsum(x, srcp, dstp, zrows)
    h = _conv0(aggs, x, W_rel1, b_rel1_2, W_root1)
    for _ in range(LAYERS):
        aggs = _segsum(h, srcp, dstp, zrows)
        h = _convr(aggs, h, W_rel2, b_rel2_2, W_root2)
    return _pool(h, batch3, W_lin, b_lin_2)


# speedup vs baseline: 2.2410x; 2.2410x over previous
import functools

import jax
import jax.numpy as jnp
from jax import lax
from jax.experimental import pallas as pl
from jax.experimental.pallas import tpu as pltpu
from jax.experimental.pallas import tpu_sc as plsc

N = 10000
E = 320000
D = 128
H = 128
C = 10
B = 64
LAYERS = 3

NC = 2
NS = 16
NW = NC * NS
EPW = E // NW
K = 80
EPWP = EPW
NCHUNK = EPWP // K
RPS = 632
NP = NS * RPS


def _segsum_body(h_hbm, src_hbm, dst_hbm, zeros_hbm, out_hbm,
                 sidx_v, didx_v, rows_v, agg_s, sem):
    c = lax.axis_index("c")
    s = lax.axis_index("s")
    w = c * NS + s

    pltpu.sync_copy(zeros_hbm, agg_s.at[pl.ds(s * RPS, RPS)])
    pltpu.sync_copy(src_hbm.at[w], sidx_v)
    pltpu.sync_copy(dst_hbm.at[w], didx_v)

    plsc.subcore_barrier()

    def eloop(j, carry):
        pltpu.async_copy(h_hbm.at[sidx_v.at[j]], rows_v, sem).wait()
        pltpu.sync_copy(rows_v, agg_s.at[didx_v.at[j]], add=True)
        return carry

    lax.fori_loop(0, NCHUNK, eloop, 0)

    plsc.subcore_barrier()

    pltpu.sync_copy(agg_s.at[pl.ds(s * RPS, RPS)],
                    out_hbm.at[c].at[pl.ds(s * RPS, RPS)])


_segsum = pl.kernel(
    _segsum_body,
    out_type=jax.ShapeDtypeStruct((NC, NP, D), jnp.float32),
    mesh=plsc.VectorSubcoreMesh(core_axis_name="c", subcore_axis_name="s"),
    scratch_types=[
        pltpu.VMEM((NCHUNK, K), jnp.int32),
        pltpu.VMEM((NCHUNK, K), jnp.int32),
        pltpu.VMEM((K, D), jnp.float32),
        pltpu.VMEM_SHARED((NP, D), jnp.float32),
        pltpu.SemaphoreType.DMA,
    ],
)


RBLK = 400
NBLK = N // RBLK

_DN_T = (((1,), (1,)), ((), ()))
_HI = lax.Precision.HIGHEST


def _conv_body(residual, aggs_ref, h_ref, wrel_ref, brel_ref, wroot_ref,
               out_ref):
    a = aggs_ref[0] + aggs_ref[1]
    t = lax.dot_general(a, wrel_ref[...], _DN_T,
                        preferred_element_type=jnp.float32, precision=_HI)
    r = lax.dot_general(h_ref[...], wroot_ref[...], _DN_T,
                        preferred_element_type=jnp.float32, precision=_HI)
    o = jnp.maximum(t + brel_ref[...] + r, 0.0)
    if residual:
        o = o + h_ref[...]
    out_ref[...] = o


def _make_conv(residual):
    return pl.pallas_call(
        functools.partial(_conv_body, residual),
        grid=(NBLK,),
        in_specs=[
            pl.BlockSpec((NC, RBLK, D), lambda i: (0, i, 0)),
            pl.BlockSpec((RBLK, D), lambda i: (i, 0)),
            pl.BlockSpec((H, D), lambda i: (0, 0)),
            pl.BlockSpec((1, H), lambda i: (0, 0)),
            pl.BlockSpec((H, D), lambda i: (0, 0)),
        ],
        out_specs=pl.BlockSpec((RBLK, H), lambda i: (i, 0)),
        out_shape=jax.ShapeDtypeStruct((N, H), jnp.float32),
    )


_conv0 = _make_conv(False)
_convr = _make_conv(True)


def _pool_body(h_ref, b3_ref, wlin_ref, blin_ref, out_ref, acc, cnt):
    i = pl.program_id(0)

    @pl.when(i == 0)
    def _():
        acc[...] = jnp.zeros_like(acc)
        cnt[...] = jnp.zeros_like(cnt)

    bid = b3_ref[0]
    rows = lax.broadcasted_iota(jnp.int32, (B, RBLK), 0)
    m = (bid == rows).astype(jnp.float32)
    acc[...] += lax.dot_general(m, h_ref[...], (((1,), (0,)), ((), ())),
                                preferred_element_type=jnp.float32,
                                precision=_HI)
    cnt[...] += jnp.broadcast_to(jnp.sum(m, axis=1, keepdims=True), (B, H))

    @pl.when(i == NBLK - 1)
    def _():
        pooled = acc[...] / jnp.maximum(cnt[...], 1.0)
        out_ref[...] = lax.dot_general(
            pooled, wlin_ref[...], _DN_T,
            preferred_element_type=jnp.float32, precision=_HI) + blin_ref[...]


_pool = pl.pallas_call(
    _pool_body,
    grid=(NBLK,),
    in_specs=[
        pl.BlockSpec((RBLK, H), lambda i: (i, 0)),
        pl.BlockSpec((1, 1, RBLK), lambda i: (i, 0, 0)),
        pl.BlockSpec((C, H), lambda i: (0, 0)),
        pl.BlockSpec((1, C), lambda i: (0, 0)),
    ],
    out_specs=pl.BlockSpec((B, C), lambda i: (0, 0)),
    out_shape=jax.ShapeDtypeStruct((B, C), jnp.float32),
    scratch_shapes=[
        pltpu.VMEM((B, H), jnp.float32),
        pltpu.VMEM((B, H), jnp.float32),
    ],
)


def kernel(x, edge_index, batch, W_rel1, b_rel1, W_root1, W_rel2, b_rel2,
           W_root2, W_lin, b_lin):
    src = edge_index[0].reshape(NW, EPW)
    dst = edge_index[1].reshape(NW, EPW)
    pad = EPWP - EPW
    srcp = jnp.concatenate(
        [src, jnp.zeros((NW, pad), jnp.int32)],
        axis=1).reshape(NW, NCHUNK, K)
    dstp = jnp.concatenate(
        [dst, jnp.full((NW, pad), N, jnp.int32)],
        axis=1).reshape(NW, NCHUNK, K)
    zrows = jnp.zeros((RPS, D), jnp.float32)
    batch3 = batch.reshape(NBLK, 1, RBLK)
    b_rel1_2 = b_rel1.reshape(1, H)
    b_rel2_2 = b_rel2.reshape(1, H)
    b_lin_2 = b_lin.reshape(1, C)

    aggs = _segsum(x, srcp, dstp, zrows)
    h = _conv0(aggs, x, W_rel1, b_rel1_2, W_root1)
    for _ in range(LAYERS):
        aggs = _segsum(h, srcp, dstp, zrows)
        h = _convr(aggs, h, W_rel2, b_rel2_2, W_root2)
    return _pool(h, batch3, W_lin, b_lin_2)


# speedup vs baseline: 2.2717x; 1.0137x over previous
import functools

import jax
import jax.numpy as jnp
from jax import lax
from jax.experimental import pallas as pl
from jax.experimental.pallas import tpu as pltpu
from jax.experimental.pallas import tpu_sc as plsc

N = 10000
E = 320000
D = 128
H = 128
C = 10
B = 64
LAYERS = 3

NC = 2
NS = 16
NW = NC * NS
EPW = E // NW
K = 80
EPWP = EPW
NCHUNK = EPWP // K
RPS = 632
NP = NS * RPS


def _segsum_body(h_hbm, src_hbm, dst_hbm, zeros_hbm, out_hbm,
                 sidx_v, didx_v, rows_v, agg_s, sem):
    c = lax.axis_index("c")
    s = lax.axis_index("s")
    w = c * NS + s

    pltpu.sync_copy(zeros_hbm, agg_s.at[pl.ds(s * RPS, RPS)])
    pltpu.sync_copy(src_hbm.at[w], sidx_v)
    pltpu.sync_copy(dst_hbm.at[w], didx_v)

    plsc.subcore_barrier()

    def eloop(j, carry):
        pltpu.async_copy(h_hbm.at[sidx_v.at[j]], rows_v, sem).wait()
        pltpu.sync_copy(rows_v, agg_s.at[didx_v.at[j]], add=True)
        return carry

    lax.fori_loop(0, NCHUNK, eloop, 0)

    plsc.subcore_barrier()

    pltpu.sync_copy(agg_s.at[pl.ds(s * RPS, RPS)],
                    out_hbm.at[c].at[pl.ds(s * RPS, RPS)])


_segsum = pl.kernel(
    _segsum_body,
    out_type=jax.ShapeDtypeStruct((NC, NP, D), jnp.float32),
    mesh=plsc.VectorSubcoreMesh(core_axis_name="c", subcore_axis_name="s"),
    scratch_types=[
        pltpu.VMEM((NCHUNK, K), jnp.int32),
        pltpu.VMEM((NCHUNK, K), jnp.int32),
        pltpu.VMEM((K, D), jnp.float32),
        pltpu.VMEM_SHARED((NP, D), jnp.float32),
        pltpu.SemaphoreType.DMA,
    ],
)


RBLK = 400
NBLK = N // RBLK

_DN_T = (((1,), (1,)), ((), ()))
_HI = lax.Precision.HIGHEST


def _conv_body(residual, aggs_ref, h_ref, wrel_ref, brel_ref, wroot_ref,
               out_ref):
    a = aggs_ref[0] + aggs_ref[1]
    t = lax.dot_general(a, wrel_ref[...], _DN_T,
                        preferred_element_type=jnp.float32, precision=_HI)
    r = lax.dot_general(h_ref[...], wroot_ref[...], _DN_T,
                        preferred_element_type=jnp.float32, precision=_HI)
    o = jnp.maximum(t + brel_ref[...] + r, 0.0)
    if residual:
        o = o + h_ref[...]
    out_ref[...] = o


def _make_conv(residual):
    return pl.pallas_call(
        functools.partial(_conv_body, residual),
        grid=(NBLK,),
        in_specs=[
            pl.BlockSpec((NC, RBLK, D), lambda i: (0, i, 0)),
            pl.BlockSpec((RBLK, D), lambda i: (i, 0)),
            pl.BlockSpec((H, D), lambda i: (0, 0)),
            pl.BlockSpec((1, H), lambda i: (0, 0)),
            pl.BlockSpec((H, D), lambda i: (0, 0)),
        ],
        out_specs=pl.BlockSpec((RBLK, H), lambda i: (i, 0)),
        out_shape=jax.ShapeDtypeStruct((N, H), jnp.float32),
    )


_conv0 = _make_conv(False)
_convr = _make_conv(True)


def _pool_body(h_ref, b3_ref, wlin_ref, blin_ref, out_ref, acc, cnt):
    i = pl.program_id(0)

    @pl.when(i == 0)
    def _():
        acc[...] = jnp.zeros_like(acc)
        cnt[...] = jnp.zeros_like(cnt)

    bid = b3_ref[0]
    rows = lax.broadcasted_iota(jnp.int32, (B, RBLK), 0)
    m = (bid == rows).astype(jnp.float32)
    acc[...] += lax.dot_general(m, h_ref[...], (((1,), (0,)), ((), ())),
                                preferred_element_type=jnp.float32,
                                precision=_HI)
    cnt[...] += jnp.broadcast_to(jnp.sum(m, axis=1, keepdims=True), (B, H))

    @pl.when(i == NBLK - 1)
    def _():
        pooled = acc[...] / jnp.maximum(cnt[...], 1.0)
        out_ref[...] = lax.dot_general(
            pooled, wlin_ref[...], _DN_T,
            preferred_element_type=jnp.float32, precision=_HI) + blin_ref[...]


_pool = pl.pallas_call(
    _pool_body,
    grid=(NBLK,),
    in_specs=[
        pl.BlockSpec((RBLK, H), lambda i: (i, 0)),
        pl.BlockSpec((1, 1, RBLK), lambda i: (i, 0, 0)),
        pl.BlockSpec((C, H), lambda i: (0, 0)),
        pl.BlockSpec((1, C), lambda i: (0, 0)),
    ],
    out_specs=pl.BlockSpec((B, C), lambda i: (0, 0)),
    out_shape=jax.ShapeDtypeStruct((B, C), jnp.float32),
    scratch_shapes=[
        pltpu.VMEM((B, H), jnp.float32),
        pltpu.VMEM((B, H), jnp.float32),
    ],
)


def _convpool_body(aggs_ref, h_ref, wrel_ref, brel_ref, wroot_ref, b3_ref,
                   wlin_ref, blin_ref, out_ref, acc, cnt):
    i = pl.program_id(0)

    @pl.when(i == 0)
    def _():
        acc[...] = jnp.zeros_like(acc)
        cnt[...] = jnp.zeros_like(cnt)

    a = aggs_ref[0] + aggs_ref[1]
    t = lax.dot_general(a, wrel_ref[...], _DN_T,
                        preferred_element_type=jnp.float32, precision=_HI)
    r = lax.dot_general(h_ref[...], wroot_ref[...], _DN_T,
                        preferred_element_type=jnp.float32, precision=_HI)
    o = jnp.maximum(t + brel_ref[...] + r, 0.0) + h_ref[...]

    bid = b3_ref[0]
    rows = lax.broadcasted_iota(jnp.int32, (B, RBLK), 0)
    m = (bid == rows).astype(jnp.float32)
    acc[...] += lax.dot_general(m, o, (((1,), (0,)), ((), ())),
                                preferred_element_type=jnp.float32,
                                precision=_HI)
    cnt[...] += jnp.broadcast_to(jnp.sum(m, axis=1, keepdims=True), (B, H))

    @pl.when(i == NBLK - 1)
    def _():
        pooled = acc[...] / jnp.maximum(cnt[...], 1.0)
        out_ref[...] = lax.dot_general(
            pooled, wlin_ref[...], _DN_T,
            preferred_element_type=jnp.float32, precision=_HI) + blin_ref[...]


_convpool = pl.pallas_call(
    _convpool_body,
    grid=(NBLK,),
    in_specs=[
        pl.BlockSpec((NC, RBLK, D), lambda i: (0, i, 0)),
        pl.BlockSpec((RBLK, D), lambda i: (i, 0)),
        pl.BlockSpec((H, D), lambda i: (0, 0)),
        pl.BlockSpec((1, H), lambda i: (0, 0)),
        pl.BlockSpec((H, D), lambda i: (0, 0)),
        pl.BlockSpec((1, 1, RBLK), lambda i: (i, 0, 0)),
        pl.BlockSpec((C, H), lambda i: (0, 0)),
        pl.BlockSpec((1, C), lambda i: (0, 0)),
    ],
    out_specs=pl.BlockSpec((B, C), lambda i: (0, 0)),
    out_shape=jax.ShapeDtypeStruct((B, C), jnp.float32),
    scratch_shapes=[
        pltpu.VMEM((B, H), jnp.float32),
        pltpu.VMEM((B, H), jnp.float32),
    ],
)


def kernel(x, edge_index, batch, W_rel1, b_rel1, W_root1, W_rel2, b_rel2,
           W_root2, W_lin, b_lin):
    src = edge_index[0].reshape(NW, EPW)
    dst = edge_index[1].reshape(NW, EPW)
    pad = EPWP - EPW
    srcp = jnp.concatenate(
        [src, jnp.zeros((NW, pad), jnp.int32)],
        axis=1).reshape(NW, NCHUNK, K)
    dstp = jnp.concatenate(
        [dst, jnp.full((NW, pad), N, jnp.int32)],
        axis=1).reshape(NW, NCHUNK, K)
    zrows = jnp.zeros((RPS, D), jnp.float32)
    batch3 = batch.reshape(NBLK, 1, RBLK)
    b_rel1_2 = b_rel1.reshape(1, H)
    b_rel2_2 = b_rel2.reshape(1, H)
    b_lin_2 = b_lin.reshape(1, C)

    aggs = _segsum(x, srcp, dstp, zrows)
    h = _conv0(aggs, x, W_rel1, b_rel1_2, W_root1)
    for _ in range(LAYERS - 1):
        aggs = _segsum(h, srcp, dstp, zrows)
        h = _convr(aggs, h, W_rel2, b_rel2_2, W_root2)
    aggs = _segsum(h, srcp, dstp, zrows)
    return _convpool(aggs, h, W_rel2, b_rel2_2, W_root2, batch3, W_lin,
                     b_lin_2)
